# baseline (speedup 1.0000x reference)
import functools

import jax
import jax.numpy as jnp
from jax import lax
from jax.experimental import pallas as pl
from jax.experimental.pallas import tpu as pltpu
from jax.experimental.pallas import tpu_sc as plsc

N = 10000
D = 128
C = 16
E = 320000
NC = 2
NS = 16
NW = NC * NS
EPW = E // NW
CHUNK = 80
NCHUNK = EPW // CHUNK
IBLK = 25
NIBLK = NCHUNK // IBLK
WIN = 8
RPT = 632
RPT_LAST = N - (NS - 1) * RPT
BLK = 1024
GRID = (N + BLK - 1) // BLK

_mesh = plsc.VectorSubcoreMesh(core_axis_name="c", subcore_axis_name="s")
_f32 = jnp.float32


@functools.partial(
    pl.kernel,
    out_type=[jax.ShapeDtypeStruct((NC, N), _f32),
              jax.ShapeDtypeStruct((NC, N), _f32)],
    mesh=_mesh,
    scratch_types=[
        pltpu.VMEM((NIBLK, IBLK, CHUNK), jnp.int32),
        pltpu.VMEM((NIBLK, IBLK, CHUNK), jnp.int32),
        pltpu.VMEM((CHUNK,), _f32),
        pltpu.VMEM_SHARED((N,), _f32),
        pltpu.VMEM_SHARED((N,), _f32),
        pltpu.SemaphoreType.DMA,
        pltpu.SemaphoreType.DMA,
    ],
    compiler_params=pltpu.CompilerParams(needs_layout_passes=False),
)
def _deg_kernel(src_hbm, dst_hbm, z1_hbm, do_hbm, di_hbm,
                src_v, dst_v, ones_v, do_sh, di_sh, sem_a, sem_b):
    cid = lax.axis_index("c")
    sid = lax.axis_index("s")
    wid = cid * NS + sid
    pltpu.sync_copy(src_hbm.at[wid], src_v)
    pltpu.sync_copy(dst_hbm.at[wid], dst_v)

    def _init_ones(k, carry):
        ones_v[pl.ds(k * 16, 16)] = jnp.ones((16,), _f32)
        return carry
    lax.fori_loop(0, CHUNK // 16, _init_ones, 0)

    @pl.when(sid == 0)
    def _():
        pltpu.sync_copy(z1_hbm, do_sh)
        pltpu.sync_copy(z1_hbm, di_sh)
    plsc.subcore_barrier()

    def _step(j, carry):
        b = j // IBLK
        k = lax.rem(j, IBLK)
        pltpu.sync_copy(ones_v, do_sh.at[src_v.at[b, k]], add=True)
        pltpu.sync_copy(ones_v, di_sh.at[dst_v.at[b, k]], add=True)
        return carry
    lax.fori_loop(0, NCHUNK, _step, 0)
    plsc.subcore_barrier()

    @pl.when(sid == 0)
    def _():
        pltpu.sync_copy(do_sh, do_hbm.at[cid])
        pltpu.sync_copy(di_sh, di_hbm.at[cid])


@functools.partial(
    pl.kernel,
    out_type=[jax.ShapeDtypeStruct((NC, N, D), _f32),
              jax.ShapeDtypeStruct((NC, N), _f32)],
    mesh=_mesh,
    scratch_types=[
        pltpu.VMEM((1, IBLK, CHUNK), jnp.int32),
        pltpu.VMEM((1, IBLK, CHUNK), jnp.int32),
        pltpu.VMEM((1, CHUNK, D), _f32),
        pltpu.VMEM((1, CHUNK), _f32),
        pltpu.VMEM_SHARED((N, D), _f32),
        pltpu.VMEM_SHARED((N,), _f32),
        pltpu.VMEM_SHARED((N,), _f32),
        pltpu.SemaphoreType.DMA((2,)),
        pltpu.SemaphoreType.DMA((2,)),
        pltpu.SemaphoreType.DMA((2,)),
        pltpu.SemaphoreType.DMA((2,)),
        pltpu.SemaphoreType.DMA,
        pltpu.SemaphoreType.DMA,
    ],
    compiler_params=pltpu.CompilerParams(needs_layout_passes=False),
)
def _agg_kernel(src_hbm, dst_hbm, y_hbm, nd_hbm, z1_hbm, z2_hbm,
                agg_hbm, c_hbm,
                src_w, dst_w, rows_v, cupd_v,
                agg_sh, c_sh, nd_sh,
                gr_sem, gc_sem, sr_sem, sc_sem, ip_src, ip_dst):
    cid = lax.axis_index("c")
    sid = lax.axis_index("s")
    wid = cid * NS + sid
    pltpu.sync_copy(src_hbm.at[wid, 0], src_w.at[0])
    pltpu.sync_copy(dst_hbm.at[wid, 0], dst_w.at[0])
    @pl.when(sid < NS - 1)
    def _():
        pltpu.sync_copy(z2_hbm.at[pl.ds(sid * RPT, RPT)],
                        agg_sh.at[pl.ds(sid * RPT, RPT)])

    @pl.when(sid == NS - 1)
    def _():
        pltpu.sync_copy(z2_hbm.at[pl.ds((NS - 1) * RPT, RPT_LAST)],
                        agg_sh.at[pl.ds((NS - 1) * RPT, RPT_LAST)])

    @pl.when(sid == 0)
    def _():
        pltpu.sync_copy(z1_hbm, c_sh)
        pltpu.sync_copy(nd_hbm, nd_sh)
    plsc.subcore_barrier()

    def _src_row(j):
        return src_w.at[0, lax.rem(j, IBLK)]

    def _dst_row(j):
        return dst_w.at[0, lax.rem(j, IBLK)]

    def _issue_gathers(j, slot):
        pltpu.async_copy(y_hbm.at[_src_row(j)], rows_v.at[slot],
                         gr_sem.at[slot])
        pltpu.async_copy(nd_hbm.at[_dst_row(j)], cupd_v.at[slot],
                         gc_sem.at[slot])

    def _wait_gathers(j, slot):
        pltpu.make_async_copy(y_hbm.at[_src_row(j)], rows_v.at[slot],
                              gr_sem.at[slot]).wait()
        pltpu.make_async_copy(nd_hbm.at[_dst_row(j)], cupd_v.at[slot],
                              gc_sem.at[slot]).wait()

    def _issue_scatters(j, slot):
        pltpu.sync_copy(rows_v.at[slot], agg_sh.at[_dst_row(j)], add=True)
        pltpu.sync_copy(cupd_v.at[slot], c_sh.at[_src_row(j)], add=True)

    def _step(j, carry):
        slot = 0
        b = j // IBLK
        k = lax.rem(j, IBLK)

        @pl.when((k == 0) & (j > 0))
        def _():
            pltpu.sync_copy(src_hbm.at[wid, b], src_w.at[0])
            pltpu.sync_copy(dst_hbm.at[wid, b], dst_w.at[0])

        _issue_gathers(j, slot)
        _wait_gathers(j, slot)
        _issue_scatters(j, slot)
        return carry
    lax.fori_loop(0, NCHUNK, _step, 0)
    plsc.subcore_barrier()

    @pl.when(sid < NS - 1)
    def _():
        pltpu.sync_copy(agg_sh.at[pl.ds(sid * RPT, RPT)],
                        agg_hbm.at[cid, pl.ds(sid * RPT, RPT)])

    @pl.when(sid == NS - 1)
    def _():
        pltpu.sync_copy(agg_sh.at[pl.ds((NS - 1) * RPT, RPT_LAST)],
                        agg_hbm.at[cid, pl.ds((NS - 1) * RPT, RPT_LAST)])

    @pl.when(sid == 0)
    def _():
        pltpu.sync_copy(c_sh, c_hbm.at[cid])


def _norms_body(dop_ref, dip_ref, ns_ref, nd_ref):
    do = dop_ref[0:1, :] + dop_ref[1:2, :]
    di = dip_ref[0:1, :] + dip_ref[1:2, :]
    ns_ref[...] = lax.rsqrt(jnp.maximum(do, 1.0))
    nd_ref[...] = lax.rsqrt(jnp.maximum(di, 1.0))


def _mm_body(x_ref, w_ref, ns_ref, y_ref):
    y_ref[...] = jnp.dot(x_ref[...], w_ref[...],
                         preferred_element_type=_f32) * ns_ref[...]


def _fin_body(agg_ref, nd_ref, ns_ref, cp_ref, b1_ref, w2_ref, b2_ref,
              out_ref, acc_ref):
    i = pl.program_id(0)

    @pl.when(i == 0)
    def _():
        acc_ref[...] = jnp.zeros_like(acc_ref)

    agg = agg_ref[0] + agg_ref[1]
    h1 = jnp.maximum(agg * nd_ref[...] + b1_ref[...], 0.0)
    c = ns_ref[...] * (cp_ref[0] + cp_ref[1])
    rows = i * BLK + lax.broadcasted_iota(jnp.int32, (BLK, 1), 0)
    contrib = jnp.where(rows < N, h1 * c, 0.0)
    acc_ref[...] += jnp.sum(contrib, axis=0, keepdims=True)

    @pl.when(i == pl.num_programs(0) - 1)
    def _():
        v = acc_ref[...] * (1.0 / N)
        out_ref[...] = jnp.dot(v, w2_ref[...],
                               preferred_element_type=_f32) + b2_ref[...]


def kernel(x, edge_index, W1, b1, W2, b2):
    src = edge_index[0].astype(jnp.int32).reshape(NW, NIBLK, IBLK, CHUNK)
    dst = edge_index[1].astype(jnp.int32).reshape(NW, NIBLK, IBLK, CHUNK)
    z1 = jnp.zeros((N,), _f32)
    z2 = jnp.zeros((N, D), _f32)

    do_p, di_p = _deg_kernel(src, dst, z1)

    ns_row, nd_row = pl.pallas_call(
        _norms_body,
        out_shape=[jax.ShapeDtypeStruct((1, N), _f32),
                   jax.ShapeDtypeStruct((1, N), _f32)],
    )(do_p, di_p)
    ns_col = ns_row.reshape(N, 1)
    nd_col = nd_row.reshape(N, 1)
    nd_flat = nd_row.reshape(N)

    y = pl.pallas_call(
        _mm_body,
        grid=(GRID,),
        in_specs=[
            pl.BlockSpec((BLK, D), lambda i: (i, 0)),
            pl.BlockSpec((D, D), lambda i: (0, 0)),
            pl.BlockSpec((BLK, 1), lambda i: (i, 0)),
        ],
        out_specs=pl.BlockSpec((BLK, D), lambda i: (i, 0)),
        out_shape=jax.ShapeDtypeStruct((N, D), _f32),
    )(x, W1, ns_col)

    agg_p, c_p = _agg_kernel(src, dst, y, nd_flat, z1, z2)
    c_p3 = c_p.reshape(NC, N, 1)

    out = pl.pallas_call(
        _fin_body,
        grid=(GRID,),
        in_specs=[
            pl.BlockSpec((NC, BLK, D), lambda i: (0, i, 0)),
            pl.BlockSpec((BLK, 1), lambda i: (i, 0)),
            pl.BlockSpec((BLK, 1), lambda i: (i, 0)),
            pl.BlockSpec((NC, BLK, 1), lambda i: (0, i, 0)),
            pl.BlockSpec((1, D), lambda i: (0, 0)),
            pl.BlockSpec((D, C), lambda i: (0, 0)),
            pl.BlockSpec((1, C), lambda i: (0, 0)),
        ],
        out_specs=pl.BlockSpec((1, C), lambda i: (0, 0)),
        out_shape=jax.ShapeDtypeStruct((1, C), _f32),
        scratch_shapes=[pltpu.VMEM((1, D), _f32)],
    )(agg_p, nd_col, ns_col, c_p3, b1.reshape(1, D), W2, b2.reshape(1, C))

    return out.reshape(C)


# speedup vs baseline: 1.1481x; 1.1481x over previous
import functools

import jax
import jax.numpy as jnp
from jax import lax
from jax.experimental import pallas as pl
from jax.experimental.pallas import tpu as pltpu
from jax.experimental.pallas import tpu_sc as plsc

N = 10000
D = 128
C = 16
E = 320000
NC = 2
NS = 16
NW = NC * NS
EPW = E // NW
CHUNK = 80
NCHUNK_A = EPW // CHUNK
EPT = E // NS
DH = D // NC
NCHUNK = EPT // CHUNK
IBLK = 10
NIBLK = NCHUNK // IBLK
RPT = 632
RPT_LAST = N - (NS - 1) * RPT
BLK = 1024
GRID = (N + BLK - 1) // BLK

_mesh = plsc.VectorSubcoreMesh(core_axis_name="c", subcore_axis_name="s")
_f32 = jnp.float32


@functools.partial(
    pl.kernel,
    out_type=[jax.ShapeDtypeStruct((NC, N), _f32),
              jax.ShapeDtypeStruct((NC, N), _f32)],
    mesh=_mesh,
    scratch_types=[
        pltpu.VMEM((NCHUNK_A, CHUNK), jnp.int32),
        pltpu.VMEM((NCHUNK_A, CHUNK), jnp.int32),
        pltpu.VMEM((CHUNK,), _f32),
        pltpu.VMEM_SHARED((N,), _f32),
        pltpu.VMEM_SHARED((N,), _f32),
        pltpu.SemaphoreType.DMA,
        pltpu.SemaphoreType.DMA,
    ],
    compiler_params=pltpu.CompilerParams(needs_layout_passes=False),
)
def _deg_kernel(src_hbm, dst_hbm, z1_hbm, do_hbm, di_hbm,
                src_v, dst_v, ones_v, do_sh, di_sh, sem_a, sem_b):
    cid = lax.axis_index("c")
    sid = lax.axis_index("s")
    wid = cid * NS + sid
    pltpu.sync_copy(src_hbm.at[wid], src_v)
    pltpu.sync_copy(dst_hbm.at[wid], dst_v)

    def _init_ones(k, carry):
        ones_v[pl.ds(k * 16, 16)] = jnp.ones((16,), _f32)
        return carry
    lax.fori_loop(0, CHUNK // 16, _init_ones, 0)

    @pl.when(sid == 0)
    def _():
        pltpu.sync_copy(z1_hbm, do_sh)
        pltpu.sync_copy(z1_hbm, di_sh)
    plsc.subcore_barrier()

    def _step(j, carry):
        pltpu.sync_copy(ones_v, do_sh.at[src_v.at[j]], add=True)
        pltpu.sync_copy(ones_v, di_sh.at[dst_v.at[j]], add=True)
        return carry
    lax.fori_loop(0, NCHUNK_A, _step, 0)
    plsc.subcore_barrier()

    @pl.when(sid == 0)
    def _():
        pltpu.sync_copy(do_sh, do_hbm.at[cid])
        pltpu.sync_copy(di_sh, di_hbm.at[cid])


@functools.partial(
    pl.kernel,
    out_type=[jax.ShapeDtypeStruct((NC, N, DH), _f32),
              jax.ShapeDtypeStruct((NC, N), _f32)],
    mesh=_mesh,
    scratch_types=[
        pltpu.VMEM((2, IBLK, CHUNK), jnp.int32),
        pltpu.VMEM((2, IBLK, CHUNK), jnp.int32),
        pltpu.VMEM((2, CHUNK, DH), _f32),
        pltpu.VMEM((2, CHUNK), _f32),
        pltpu.VMEM_SHARED((N, DH), _f32),
        pltpu.VMEM_SHARED((N,), _f32),
        pltpu.VMEM_SHARED((N,), _f32),
        pltpu.SemaphoreType.DMA((2,)),
        pltpu.SemaphoreType.DMA((2,)),
        pltpu.SemaphoreType.DMA((2,)),
        pltpu.SemaphoreType.DMA((2,)),
        pltpu.SemaphoreType.DMA,
        pltpu.SemaphoreType.DMA,
    ],
    compiler_params=pltpu.CompilerParams(needs_layout_passes=False,
                                         use_tc_tiling_on_sc=False),
)
def _agg_kernel(src_hbm, dst_hbm, y_hbm, nd_hbm, z1_hbm, z2_hbm,
                agg_hbm, c_hbm,
                src_w, dst_w, rows_v, cupd_v,
                agg_sh, c_sh, nd_sh,
                gr_sem, gc_sem, sr_sem, sc_sem, ip_src, ip_dst):
    cid = lax.axis_index("c")
    sid = lax.axis_index("s")
    yh = y_hbm.at[cid]
    pltpu.sync_copy(src_hbm.at[sid, 0], src_w.at[0])
    pltpu.sync_copy(dst_hbm.at[sid, 0], dst_w.at[0])
    @pl.when(sid < NS - 1)
    def _():
        pltpu.sync_copy(z2_hbm.at[pl.ds(sid * RPT, RPT)],
                        agg_sh.at[pl.ds(sid * RPT, RPT)])

    @pl.when(sid == NS - 1)
    def _():
        pltpu.sync_copy(z2_hbm.at[pl.ds((NS - 1) * RPT, RPT_LAST)],
                        agg_sh.at[pl.ds((NS - 1) * RPT, RPT_LAST)])

    @pl.when(sid == 0)
    def _():
        pltpu.sync_copy(z1_hbm, c_sh)
        pltpu.sync_copy(nd_hbm, nd_sh)
    plsc.subcore_barrier()

    def _src_row(j):
        return src_w.at[lax.rem(j // IBLK, 2), lax.rem(j, IBLK)]

    def _dst_row(j):
        return dst_w.at[lax.rem(j // IBLK, 2), lax.rem(j, IBLK)]

    def _issue_gathers(j, slot):
        pltpu.async_copy(yh.at[_src_row(j)], rows_v.at[slot],
                         gr_sem.at[slot])
        pltpu.async_copy(nd_sh.at[_dst_row(j)], cupd_v.at[slot],
                         gc_sem.at[slot])

    def _wait_gathers(j, slot):
        pltpu.make_async_copy(yh.at[_src_row(j)], rows_v.at[slot],
                              gr_sem.at[slot]).wait()
        pltpu.make_async_copy(nd_sh.at[_dst_row(j)], cupd_v.at[slot],
                              gc_sem.at[slot]).wait()

    def _issue_scatters(j, slot):
        pltpu.async_copy(rows_v.at[slot], agg_sh.at[_dst_row(j)],
                         sr_sem.at[slot], add=True)
        pltpu.async_copy(cupd_v.at[slot], c_sh.at[_src_row(j)],
                         sc_sem.at[slot], add=True)

    def _wait_scatters(j, slot):
        pltpu.make_async_copy(rows_v.at[slot], agg_sh.at[_dst_row(j)],
                              sr_sem.at[slot]).wait()
        pltpu.make_async_copy(cupd_v.at[slot], c_sh.at[_src_row(j)],
                              sc_sem.at[slot]).wait()

    _issue_gathers(0, 0)

    def _step(j, carry):
        slot = lax.rem(j, 2)
        nslot = 1 - slot
        b = j // IBLK
        k = lax.rem(j, IBLK)

        @pl.when(j >= 1)
        def _():
            _wait_scatters(j - 1, nslot)

        @pl.when((k == 0) & (b + 1 < NIBLK))
        def _():
            pltpu.async_copy(src_hbm.at[sid, b + 1],
                             src_w.at[lax.rem(b + 1, 2)], ip_src)
            pltpu.async_copy(dst_hbm.at[sid, b + 1],
                             dst_w.at[lax.rem(b + 1, 2)], ip_dst)

        @pl.when((k == IBLK - 1) & (b + 1 < NIBLK))
        def _():
            pltpu.make_async_copy(src_hbm.at[sid, 0],
                                  src_w.at[lax.rem(b + 1, 2)],
                                  ip_src).wait()
            pltpu.make_async_copy(dst_hbm.at[sid, 0],
                                  dst_w.at[lax.rem(b + 1, 2)],
                                  ip_dst).wait()

        @pl.when(j + 1 < NCHUNK)
        def _():
            _issue_gathers(j + 1, nslot)
        _wait_gathers(j, slot)
        _issue_scatters(j, slot)
        return carry
    lax.fori_loop(0, NCHUNK, _step, 0)
    _wait_scatters(NCHUNK - 1, lax.rem(NCHUNK - 1, 2))
    plsc.subcore_barrier()

    @pl.when(sid < NS - 1)
    def _():
        pltpu.sync_copy(agg_sh.at[pl.ds(sid * RPT, RPT)],
                        agg_hbm.at[cid, pl.ds(sid * RPT, RPT)])

    @pl.when(sid == NS - 1)
    def _():
        pltpu.sync_copy(agg_sh.at[pl.ds((NS - 1) * RPT, RPT_LAST)],
                        agg_hbm.at[cid, pl.ds((NS - 1) * RPT, RPT_LAST)])

    @pl.when(sid == 0)
    def _():
        pltpu.sync_copy(c_sh, c_hbm.at[cid])


def _norms_body(dop_ref, dip_ref, ns_ref, nd_ref):
    do = dop_ref[0:1, :] + dop_ref[1:2, :]
    di = dip_ref[0:1, :] + dip_ref[1:2, :]
    ns_ref[...] = lax.rsqrt(jnp.maximum(do, 1.0))
    nd_ref[...] = lax.rsqrt(jnp.maximum(di, 1.0))


def _mm_body(x_ref, w_ref, ns_ref, y_ref):
    yy = jnp.dot(x_ref[...], w_ref[...],
                 preferred_element_type=_f32) * ns_ref[...]
    y_ref[0] = yy[:, :DH]
    y_ref[1] = yy[:, DH:]


def _fin_body(agg_ref, nd_ref, ns_ref, cp_ref, b1_ref, w2_ref, b2_ref,
              out_ref, acc_ref):
    i = pl.program_id(0)

    @pl.when(i == 0)
    def _():
        acc_ref[...] = jnp.zeros_like(acc_ref)

    agg = jnp.concatenate([agg_ref[0], agg_ref[1]], axis=1)
    h1 = jnp.maximum(agg * nd_ref[...] + b1_ref[...], 0.0)
    c = ns_ref[...] * (0.5 * (cp_ref[0] + cp_ref[1]))
    rows = i * BLK + lax.broadcasted_iota(jnp.int32, (BLK, 1), 0)
    contrib = jnp.where(rows < N, h1 * c, 0.0)
    acc_ref[...] += jnp.sum(contrib, axis=0, keepdims=True)

    @pl.when(i == pl.num_programs(0) - 1)
    def _():
        v = acc_ref[...] * (1.0 / N)
        out_ref[...] = jnp.dot(v, w2_ref[...],
                               preferred_element_type=_f32) + b2_ref[...]


def kernel(x, edge_index, W1, b1, W2, b2):
    src32 = edge_index[0].astype(jnp.int32)
    dst32 = edge_index[1].astype(jnp.int32)
    src3 = src32.reshape(NW, NCHUNK_A, CHUNK)
    dst3 = dst32.reshape(NW, NCHUNK_A, CHUNK)
    src4 = src32.reshape(NS, NIBLK, IBLK, CHUNK)
    dst4 = dst32.reshape(NS, NIBLK, IBLK, CHUNK)
    z1 = jnp.zeros((N,), _f32)
    z2 = jnp.zeros((N, DH), _f32)

    do_p, di_p = _deg_kernel(src3, dst3, z1)

    ns_row, nd_row = pl.pallas_call(
        _norms_body,
        out_shape=[jax.ShapeDtypeStruct((1, N), _f32),
                   jax.ShapeDtypeStruct((1, N), _f32)],
    )(do_p, di_p)
    ns_col = ns_row.reshape(N, 1)
    nd_col = nd_row.reshape(N, 1)
    nd_flat = nd_row.reshape(N)

    y = pl.pallas_call(
        _mm_body,
        grid=(GRID,),
        in_specs=[
            pl.BlockSpec((BLK, D), lambda i: (i, 0)),
            pl.BlockSpec((D, D), lambda i: (0, 0)),
            pl.BlockSpec((BLK, 1), lambda i: (i, 0)),
        ],
        out_specs=pl.BlockSpec((NC, BLK, DH), lambda i: (0, i, 0)),
        out_shape=jax.ShapeDtypeStruct((NC, N, DH), _f32),
    )(x, W1, ns_col)

    agg_p, c_p = _agg_kernel(src4, dst4, y, nd_flat, z1, z2)
    c_p3 = c_p.reshape(NC, N, 1)

    out = pl.pallas_call(
        _fin_body,
        grid=(GRID,),
        in_specs=[
            pl.BlockSpec((NC, BLK, DH), lambda i: (0, i, 0)),
            pl.BlockSpec((BLK, 1), lambda i: (i, 0)),
            pl.BlockSpec((BLK, 1), lambda i: (i, 0)),
            pl.BlockSpec((NC, BLK, 1), lambda i: (0, i, 0)),
            pl.BlockSpec((1, D), lambda i: (0, 0)),
            pl.BlockSpec((D, C), lambda i: (0, 0)),
            pl.BlockSpec((1, C), lambda i: (0, 0)),
        ],
        out_specs=pl.BlockSpec((1, C), lambda i: (0, 0)),
        out_shape=jax.ShapeDtypeStruct((1, C), _f32),
        scratch_shapes=[pltpu.VMEM((1, D), _f32)],
    )(agg_p, nd_col, ns_col, c_p3, b1.reshape(1, D), W2, b2.reshape(1, C))

    return out.reshape(C)


# speedup vs baseline: 1.3033x; 1.1352x over previous
import functools

import jax
import jax.numpy as jnp
from jax import lax
from jax.experimental import pallas as pl
from jax.experimental.pallas import tpu as pltpu
from jax.experimental.pallas import tpu_sc as plsc

N = 10000
D = 128
C = 16
E = 320000
NC = 2
NS = 16
NW = NC * NS
EPW = E // NW
CHUNK = 80
NCHUNK_A = EPW // CHUNK
EPT = E // NS
DH = D // NC
NCHUNK = EPT // CHUNK
IBLK = 10
NIBLK = NCHUNK // IBLK
WIN = 4
RPT = 632
RPT_LAST = N - (NS - 1) * RPT
BLK = 1024
GRID = (N + BLK - 1) // BLK

_mesh = plsc.VectorSubcoreMesh(core_axis_name="c", subcore_axis_name="s")
_f32 = jnp.float32


@functools.partial(
    pl.kernel,
    out_type=[jax.ShapeDtypeStruct((NC, N), _f32),
              jax.ShapeDtypeStruct((NC, N), _f32)],
    mesh=_mesh,
    scratch_types=[
        pltpu.VMEM((NCHUNK_A, CHUNK), jnp.int32),
        pltpu.VMEM((NCHUNK_A, CHUNK), jnp.int32),
        pltpu.VMEM((CHUNK,), _f32),
        pltpu.VMEM_SHARED((N,), _f32),
        pltpu.VMEM_SHARED((N,), _f32),
        pltpu.SemaphoreType.DMA,
        pltpu.SemaphoreType.DMA,
    ],
    compiler_params=pltpu.CompilerParams(needs_layout_passes=False),
)
def _deg_kernel(src_hbm, dst_hbm, z1_hbm, do_hbm, di_hbm,
                src_v, dst_v, ones_v, do_sh, di_sh, sem_a, sem_b):
    cid = lax.axis_index("c")
    sid = lax.axis_index("s")
    wid = cid * NS + sid
    pltpu.sync_copy(src_hbm.at[wid], src_v)
    pltpu.sync_copy(dst_hbm.at[wid], dst_v)

    def _init_ones(k, carry):
        ones_v[pl.ds(k * 16, 16)] = jnp.ones((16,), _f32)
        return carry
    lax.fori_loop(0, CHUNK // 16, _init_ones, 0)

    @pl.when(sid == 0)
    def _():
        pltpu.sync_copy(z1_hbm, do_sh)
        pltpu.sync_copy(z1_hbm, di_sh)
    plsc.subcore_barrier()

    def _step(j, carry):
        @pl.when(j >= WIN)
        def _():
            pltpu.make_async_copy(
                ones_v, do_sh.at[src_v.at[j - WIN]], sem_a).wait()
            pltpu.make_async_copy(
                ones_v, di_sh.at[dst_v.at[j - WIN]], sem_b).wait()
        pltpu.async_copy(ones_v, do_sh.at[src_v.at[j]], sem_a, add=True)
        pltpu.async_copy(ones_v, di_sh.at[dst_v.at[j]], sem_b, add=True)
        return carry
    lax.fori_loop(0, NCHUNK_A, _step, 0)

    def _drain(j, carry):
        pltpu.make_async_copy(ones_v, do_sh.at[src_v.at[j]], sem_a).wait()
        pltpu.make_async_copy(ones_v, di_sh.at[dst_v.at[j]], sem_b).wait()
        return carry
    lax.fori_loop(NCHUNK_A - WIN, NCHUNK_A, _drain, 0)
    plsc.subcore_barrier()

    @pl.when(sid == 0)
    def _():
        pltpu.sync_copy(do_sh, do_hbm.at[cid])
        pltpu.sync_copy(di_sh, di_hbm.at[cid])


@functools.partial(
    pl.kernel,
    out_type=[jax.ShapeDtypeStruct((NC, N, DH), _f32),
              jax.ShapeDtypeStruct((NC, N), _f32)],
    mesh=_mesh,
    scratch_types=[
        pltpu.VMEM((2, IBLK, CHUNK), jnp.int32),
        pltpu.VMEM((2, IBLK, CHUNK), jnp.int32),
        pltpu.VMEM((4, CHUNK, DH), _f32),
        pltpu.VMEM((4, CHUNK), _f32),
        pltpu.VMEM_SHARED((N, DH), _f32),
        pltpu.VMEM_SHARED((N,), _f32),
        pltpu.VMEM_SHARED((N,), _f32),
        pltpu.SemaphoreType.DMA((4,)),
        pltpu.SemaphoreType.DMA((4,)),
        pltpu.SemaphoreType.DMA((4,)),
        pltpu.SemaphoreType.DMA((4,)),
        pltpu.SemaphoreType.DMA,
        pltpu.SemaphoreType.DMA,
    ],
    compiler_params=pltpu.CompilerParams(needs_layout_passes=False,
                                         use_tc_tiling_on_sc=False),
)
def _agg_kernel(src_hbm, dst_hbm, y_hbm, nd_hbm, z1_hbm, z2_hbm,
                agg_hbm, c_hbm,
                src_w, dst_w, rows_v, cupd_v,
                agg_sh, c_sh, nd_sh,
                gr_sem, gc_sem, sr_sem, sc_sem, ip_src, ip_dst):
    cid = lax.axis_index("c")
    sid = lax.axis_index("s")
    yh = y_hbm.at[cid]
    pltpu.sync_copy(src_hbm.at[sid, 0], src_w.at[0])
    pltpu.sync_copy(dst_hbm.at[sid, 0], dst_w.at[0])
    @pl.when(sid < NS - 1)
    def _():
        pltpu.sync_copy(z2_hbm.at[pl.ds(sid * RPT, RPT)],
                        agg_sh.at[pl.ds(sid * RPT, RPT)])

    @pl.when(sid == NS - 1)
    def _():
        pltpu.sync_copy(z2_hbm.at[pl.ds((NS - 1) * RPT, RPT_LAST)],
                        agg_sh.at[pl.ds((NS - 1) * RPT, RPT_LAST)])

    @pl.when(sid == 0)
    def _():
        pltpu.sync_copy(z1_hbm, c_sh)
        pltpu.sync_copy(nd_hbm, nd_sh)
    plsc.subcore_barrier()

    def _src_row(j):
        return src_w.at[lax.rem(j // IBLK, 2), lax.rem(j, IBLK)]

    def _dst_row(j):
        return dst_w.at[lax.rem(j // IBLK, 2), lax.rem(j, IBLK)]

    def _issue_gathers(j, slot):
        pltpu.async_copy(yh.at[_src_row(j)], rows_v.at[slot],
                         gr_sem.at[slot])
        pltpu.async_copy(nd_sh.at[_dst_row(j)], cupd_v.at[slot],
                         gc_sem.at[slot])

    def _wait_gathers(j, slot):
        pltpu.make_async_copy(yh.at[_src_row(j)], rows_v.at[slot],
                              gr_sem.at[slot]).wait()
        pltpu.make_async_copy(nd_sh.at[_dst_row(j)], cupd_v.at[slot],
                              gc_sem.at[slot]).wait()

    def _issue_scatters(j, slot):
        pltpu.async_copy(rows_v.at[slot], agg_sh.at[_dst_row(j)],
                         sr_sem.at[slot], add=True)
        pltpu.async_copy(cupd_v.at[slot], c_sh.at[_src_row(j)],
                         sc_sem.at[slot], add=True)

    def _wait_scatters(j, slot):
        pltpu.make_async_copy(rows_v.at[slot], agg_sh.at[_dst_row(j)],
                              sr_sem.at[slot]).wait()
        pltpu.make_async_copy(cupd_v.at[slot], c_sh.at[_src_row(j)],
                              sc_sem.at[slot]).wait()

    _issue_gathers(0, 0)

    def _step(j, carry):
        slot = lax.rem(j, 4)
        nslot = lax.rem(j + 1, 4)
        b = j // IBLK
        k = lax.rem(j, IBLK)

        @pl.when(j >= 3)
        def _():
            _wait_scatters(j - 3, nslot)

        @pl.when((k == 3) & (b + 1 < NIBLK))
        def _():
            pltpu.async_copy(src_hbm.at[sid, b + 1],
                             src_w.at[lax.rem(b + 1, 2)], ip_src)
            pltpu.async_copy(dst_hbm.at[sid, b + 1],
                             dst_w.at[lax.rem(b + 1, 2)], ip_dst)

        @pl.when((k == IBLK - 1) & (b + 1 < NIBLK))
        def _():
            pltpu.make_async_copy(src_hbm.at[sid, 0],
                                  src_w.at[lax.rem(b + 1, 2)],
                                  ip_src).wait()
            pltpu.make_async_copy(dst_hbm.at[sid, 0],
                                  dst_w.at[lax.rem(b + 1, 2)],
                                  ip_dst).wait()

        @pl.when(j + 1 < NCHUNK)
        def _():
            _issue_gathers(j + 1, nslot)
        _wait_gathers(j, slot)
        _issue_scatters(j, slot)
        return carry
    lax.fori_loop(0, NCHUNK, _step, 0)

    def _draint(j, carry):
        _wait_scatters(j, lax.rem(j, 4))
        return carry
    lax.fori_loop(NCHUNK - 3, NCHUNK, _draint, 0)
    plsc.subcore_barrier()

    @pl.when(sid < NS - 1)
    def _():
        pltpu.sync_copy(agg_sh.at[pl.ds(sid * RPT, RPT)],
                        agg_hbm.at[cid, pl.ds(sid * RPT, RPT)])

    @pl.when(sid == NS - 1)
    def _():
        pltpu.sync_copy(agg_sh.at[pl.ds((NS - 1) * RPT, RPT_LAST)],
                        agg_hbm.at[cid, pl.ds((NS - 1) * RPT, RPT_LAST)])

    @pl.when(sid == 0)
    def _():
        pltpu.sync_copy(c_sh, c_hbm.at[cid])


def _norms_body(dop_ref, dip_ref, ns_ref, nd_ref):
    do = dop_ref[0:1, :] + dop_ref[1:2, :]
    di = dip_ref[0:1, :] + dip_ref[1:2, :]
    ns_ref[...] = lax.rsqrt(jnp.maximum(do, 1.0))
    nd_ref[...] = lax.rsqrt(jnp.maximum(di, 1.0))


def _mm_body(x_ref, w_ref, ns_ref, y_ref):
    yy = jnp.dot(x_ref[...], w_ref[...],
                 preferred_element_type=_f32) * ns_ref[...]
    y_ref[0] = yy[:, :DH]
    y_ref[1] = yy[:, DH:]


def _fin_body(agg_ref, nd_ref, ns_ref, cp_ref, b1_ref, w2_ref, b2_ref,
              out_ref, acc_ref):
    i = pl.program_id(0)

    @pl.when(i == 0)
    def _():
        acc_ref[...] = jnp.zeros_like(acc_ref)

    agg = jnp.concatenate([agg_ref[0], agg_ref[1]], axis=1)
    h1 = jnp.maximum(agg * nd_ref[...] + b1_ref[...], 0.0)
    c = ns_ref[...] * (0.5 * (cp_ref[0] + cp_ref[1]))
    rows = i * BLK + lax.broadcasted_iota(jnp.int32, (BLK, 1), 0)
    contrib = jnp.where(rows < N, h1 * c, 0.0)
    acc_ref[...] += jnp.sum(contrib, axis=0, keepdims=True)

    @pl.when(i == pl.num_programs(0) - 1)
    def _():
        v = acc_ref[...] * (1.0 / N)
        out_ref[...] = jnp.dot(v, w2_ref[...],
                               preferred_element_type=_f32) + b2_ref[...]


def kernel(x, edge_index, W1, b1, W2, b2):
    src32 = edge_index[0].astype(jnp.int32)
    dst32 = edge_index[1].astype(jnp.int32)
    src3 = src32.reshape(NW, NCHUNK_A, CHUNK)
    dst3 = dst32.reshape(NW, NCHUNK_A, CHUNK)
    src4 = src32.reshape(NS, NIBLK, IBLK, CHUNK)
    dst4 = dst32.reshape(NS, NIBLK, IBLK, CHUNK)
    z1 = jnp.zeros((N,), _f32)
    z2 = jnp.zeros((N, DH), _f32)

    do_p, di_p = _deg_kernel(src3, dst3, z1)

    ns_row, nd_row = pl.pallas_call(
        _norms_body,
        out_shape=[jax.ShapeDtypeStruct((1, N), _f32),
                   jax.ShapeDtypeStruct((1, N), _f32)],
    )(do_p, di_p)
    ns_col = ns_row.reshape(N, 1)
    nd_col = nd_row.reshape(N, 1)
    nd_flat = nd_row.reshape(N)

    y = pl.pallas_call(
        _mm_body,
        grid=(GRID,),
        in_specs=[
            pl.BlockSpec((BLK, D), lambda i: (i, 0)),
            pl.BlockSpec((D, D), lambda i: (0, 0)),
            pl.BlockSpec((BLK, 1), lambda i: (i, 0)),
        ],
        out_specs=pl.BlockSpec((NC, BLK, DH), lambda i: (0, i, 0)),
        out_shape=jax.ShapeDtypeStruct((NC, N, DH), _f32),
    )(x, W1, ns_col)

    agg_p, c_p = _agg_kernel(src4, dst4, y, nd_flat, z1, z2)
    c_p3 = c_p.reshape(NC, N, 1)

    out = pl.pallas_call(
        _fin_body,
        grid=(GRID,),
        in_specs=[
            pl.BlockSpec((NC, BLK, DH), lambda i: (0, i, 0)),
            pl.BlockSpec((BLK, 1), lambda i: (i, 0)),
            pl.BlockSpec((BLK, 1), lambda i: (i, 0)),
            pl.BlockSpec((NC, BLK, 1), lambda i: (0, i, 0)),
            pl.BlockSpec((1, D), lambda i: (0, 0)),
            pl.BlockSpec((D, C), lambda i: (0, 0)),
            pl.BlockSpec((1, C), lambda i: (0, 0)),
        ],
        out_specs=pl.BlockSpec((1, C), lambda i: (0, 0)),
        out_shape=jax.ShapeDtypeStruct((1, C), _f32),
        scratch_shapes=[pltpu.VMEM((1, D), _f32)],
    )(agg_p, nd_col, ns_col, c_p3, b1.reshape(1, D), W2, b2.reshape(1, C))

    return out.reshape(C)
